# Initial kernel scaffold; baseline (speedup 1.0000x reference)
import jax
import jax.numpy as jnp
from jax.experimental import pallas as pl


def kernel(x, adjs, edge_attr, W_e, b_e, W_a2, b_a2, W_l, b_l):
    raise NotImplementedError("write your pallas kernel here")


# speedup vs baseline: 1.0157x; 1.0157x over previous
import jax
import jax.numpy as jnp
from jax.experimental import pallas as pl

N_NODES = 10000
E = 320000
BLK = 128


def _inv_clean(v):
    r = v ** (-1.0)
    r = jnp.where(jnp.isnan(r), 1.0, r)
    r = jnp.where(r == jnp.inf, 1.0, r)
    return r


def _combine_body(e_ref, gmax_ref, gmean_ref, gmin_ref, gsum_ref, w_ref, out_ref):
    e = e_ref[...]
    e2 = jnp.where(e == jnp.inf, 1.0, e)
    e2 = jnp.where(jnp.isnan(e2), 1.0, e2)
    w = w_ref[...]
    nc = (w[0, 0] * _inv_clean(gmax_ref[...])
          + w[0, 1] * _inv_clean(gmean_ref[...])
          + w[0, 2] * _inv_clean(gmin_ref[...])
          + w[0, 3] * _inv_clean(gsum_ref[...])
          + w[0, 4])
    out = (e2 * w[0, 5] + w[0, 6]) + nc * e2
    out_ref[...] = jnp.where(out == jnp.inf, 1.0, out)


def kernel(x, adjs, edge_attr, W_e, b_e, W_a2, b_a2, W_l, b_l):
    e = (edge_attr @ W_e.T + b_e)
    ei = jnp.concatenate([adjs, adjs[::-1]], axis=1)
    kkey = ei[1].astype(jnp.int32) * N_NODES + ei[0].astype(jnp.int32)
    n_max = kkey.shape[0]
    uniq, inv = jnp.unique(kkey, return_inverse=True, size=n_max, fill_value=-1)
    inv = inv.reshape(-1)
    dst_u = jnp.where(uniq < 0, N_NODES, uniq // N_NODES).astype(jnp.int32)
    ea2 = jnp.concatenate([e, e], axis=0)
    undi = jax.ops.segment_sum(ea2, inv, num_segments=n_max)
    cnt = jax.ops.segment_sum(jnp.ones((n_max,), jnp.float32), dst_u,
                              num_segments=N_NODES + 1)[:N_NODES][:, None]
    nsum = jax.ops.segment_sum(undi, dst_u, num_segments=N_NODES + 1)[:N_NODES]
    nmax = jax.ops.segment_max(undi, dst_u, num_segments=N_NODES + 1)[:N_NODES]
    nmin = jax.ops.segment_min(undi, dst_u, num_segments=N_NODES + 1)[:N_NODES]
    nmax = jnp.where(cnt > 0, nmax, 0.0)
    nmin = jnp.where(cnt > 0, nmin, 0.0)
    nmean = nsum / jnp.maximum(cnt, 1.0)
    dst_e = adjs[1]
    g_max = jnp.take(nmax, dst_e, axis=0).reshape(-1, BLK)
    g_mean = jnp.take(nmean, dst_e, axis=0).reshape(-1, BLK)
    g_min = jnp.take(nmin, dst_e, axis=0).reshape(-1, BLK)
    g_sum = jnp.take(nsum, dst_e, axis=0).reshape(-1, BLK)
    w = jnp.concatenate([W_a2[0], b_a2, W_l[0], b_l, jnp.zeros((1,), jnp.float32)]
                        ).reshape(1, 8)
    e_r = e.reshape(-1, BLK)
    out = pl.pallas_call(
        _combine_body,
        out_shape=jax.ShapeDtypeStruct((E // BLK, BLK), jnp.float32),
    )(e_r, g_max, g_mean, g_min, g_sum, w)
    return out.reshape(-1)


# speedup vs baseline: 12.0439x; 11.8576x over previous
import functools

import jax
import jax.numpy as jnp
from jax import lax
from jax.experimental import pallas as pl
from jax.experimental.pallas import tpu as pltpu
from jax.experimental.pallas import tpu_sc as plsc

N = 10000
E = 320000
DF = 16

NW = 32
NB = 320
NPAD = NW * NB
CH = 3200
NCHUNK = E // CH
CAP = 21504
CAPP = CAP + 16
CAPS = CAP + 48
HB = NB + 16
CH3 = 2000
EPT = E // NW

_IOTA = None


def _lane(v, k):
    i = lax.iota(jnp.int32, 16)
    z = jnp.zeros((16,), v.dtype)
    return jnp.sum(jnp.where(i == k, v, z))


def _matmul_body(x_ref, m_ref, b_ref, o_ref):
    o_ref[...] = jnp.dot(x_ref[...], m_ref[...],
                         preferred_element_type=jnp.float32) + b_ref[0, 0]


def _edge_scalar(edge_attr, W_e, b_e):
    j = lax.broadcasted_iota(jnp.int32, (2048, 128), 0)
    l = lax.broadcasted_iota(jnp.int32, (2048, 128), 1)
    M = jnp.where(j // DF == l, W_e[0][j % DF], 0.0)
    xv = edge_attr.reshape(2500, 2048)
    b = b_e.reshape(1, 1)
    out = pl.pallas_call(
        _matmul_body,
        in_specs=[
            pl.BlockSpec((2500, 2048), lambda: (0, 0)),
            pl.BlockSpec((2048, 128), lambda: (0, 0)),
            pl.BlockSpec(memory_space=pltpu.SMEM),
        ],
        out_specs=pl.BlockSpec((2500, 128), lambda: (0, 0)),
        out_shape=jax.ShapeDtypeStruct((2500, 128), jnp.float32),
    )(xv, M, b)
    return out.reshape(E)


def _graph_body(src_hbm, dst_hbm, e_hbm, wa_hbm,
                a_out_hbm,
                a0b, a1b, eb, fd, fs, fv, ss, sv,
                hist, cursor, starts, wav, aout):
    i16 = lax.iota(jnp.int32, 16)
    z16f = jnp.zeros((16,), jnp.float32)
    z16i = jnp.zeros((16,), jnp.int32)
    wid = lax.axis_index("s") * 2 + lax.axis_index("c")
    base = wid * NB

    pltpu.sync_copy(wa_hbm, wav)
    wa = wav[...]

    def filt_chunk(c, cur, d_hbm, s_hbm):
        pltpu.sync_copy(d_hbm.at[pl.ds(c * CH, CH)], a0b)
        pltpu.sync_copy(s_hbm.at[pl.ds(c * CH, CH)], a1b)
        pltpu.sync_copy(e_hbm.at[pl.ds(c * CH, CH)], eb)

        def body(i, cur):
            d = a0b[pl.ds(i * 16, 16)]
            s = a1b[pl.ds(i * 16, 16)]
            v = eb[pl.ds(i * 16, 16)]
            rd = d - base
            m = (rd >= 0) & (rd < NB)
            pc = jnp.sum(jnp.where(m, jnp.ones((16,), jnp.float32),
                                   z16f)).astype(jnp.int32)
            cc = jnp.minimum(cur, CAP)
            plsc.store_compressed(fd.at[pl.ds(cc, 16)], rd, mask=m)
            plsc.store_compressed(fs.at[pl.ds(cc, 16)], s, mask=m)
            plsc.store_compressed(fv.at[pl.ds(cc, 16)], v, mask=m)
            return cur + pc

        return lax.fori_loop(0, CH // 16, body, cur)

    def half0(c, cur):
        return filt_chunk(c, cur, dst_hbm, src_hbm)

    def half1(c, cur):
        return filt_chunk(c, cur, src_hbm, dst_hbm)

    cur = lax.fori_loop(0, NCHUNK, half0, jnp.int32(0))
    cur = lax.fori_loop(0, NCHUNK, half1, cur)
    cur = jnp.minimum(cur, CAP)

    fd[pl.ds(cur, 16)] = jnp.full((16,), NB, jnp.int32)
    fs[pl.ds(cur, 16)] = z16i
    fv[pl.ds(cur, 16)] = z16f
    nvec = cur // 16 + 1

    def zh(i, _):
        hist[pl.ds(i * 16, 16)] = z16f
        return 0

    lax.fori_loop(0, HB // 16, zh, 0)

    def hbody(i, _):
        rd = fd[pl.ds(i * 16, 16)]
        cnt, lastm = plsc.scan_count(rd)
        plsc.addupdate_scatter(hist, [rd], cnt.astype(jnp.float32),
                               mask=lastm)
        return 0

    lax.fori_loop(0, nvec, hbody, 0)

    def scan_body(i, carry):
        h = hist[pl.ds(i * 16, 16)]
        ex = plsc.cumsum(h) - h + carry
        cursor[pl.ds(i * 16, 16)] = ex
        starts[pl.ds(i * 16, 16)] = ex
        return carry + jnp.sum(h)

    lax.fori_loop(0, HB // 16, scan_body, jnp.float32(0))

    def pbody(i, _):
        rd = fd[pl.ds(i * 16, 16)]
        s = fs[pl.ds(i * 16, 16)]
        v = fv[pl.ds(i * 16, 16)]
        cnt, lastm = plsc.scan_count(rd)
        cntf = cnt.astype(jnp.float32)
        bpos = plsc.load_gather(cursor, [rd])
        pos = (bpos + cntf).astype(jnp.int32) - 1
        plsc.store_scatter(ss, [pos], s)
        plsc.store_scatter(sv, [pos], v)
        plsc.addupdate_scatter(cursor, [rd], cntf, mask=lastm)
        return 0

    lax.fori_loop(0, nvec, pbody, 0)

    mark, claim, table = fd, fs, fv

    def zm(i, _):
        mark[pl.ds(i * 16, 16)] = jnp.full((16,), -1, jnp.int32)
        return 0

    lax.fori_loop(0, N // 16, zm, 0)

    w0 = _lane(wa, 0)
    w1 = _lane(wa, 1)
    w2 = _lane(wa, 2)
    w3 = _lane(wa, 3)
    ba2 = _lane(wa, 4)

    def ic(x):
        r = 1.0 / x
        r = jnp.where(jnp.isnan(r), 1.0, r)
        r = jnp.where(r == jnp.inf, 1.0, r)
        return r

    def node_body(nr, _):
        stv = starts[pl.ds(nr, 16)]
        s0 = _lane(stv, 0).astype(jnp.int32)
        s1 = _lane(stv, 1).astype(jnp.int32)
        ln = s1 - s0
        nv = (ln + 15) // 16
        nabs = base + nr

        def pass_a(t, _):
            off = s0 + t * 16
            m = i16 < (ln - t * 16)
            s = jnp.where(m, ss[pl.ds(off, 16)], z16i)
            _, lastm = plsc.scan_count(s, mask=m)
            um = lastm & m
            mk = plsc.load_gather(mark, [s], mask=um)
            stale = (mk != nabs) & um
            plsc.store_scatter(mark, [s], jnp.full((16,), nabs, jnp.int32),
                               mask=stale)
            plsc.store_scatter(table, [s], z16f, mask=stale)
            plsc.store_scatter(claim, [s], off + i16, mask=um)
            return 0

        lax.fori_loop(0, nv, pass_a, 0)

        def pass_b(t, _):
            off = s0 + t * 16
            m = i16 < (ln - t * 16)
            s = jnp.where(m, ss[pl.ds(off, 16)], z16i)
            v = jnp.where(m, sv[pl.ds(off, 16)], z16f)
            ks, vs, m2 = plsc.sort_key_val(s, v, mask=m)
            vs = jnp.where(m2, vs, z16f)
            csum = plsc.cumsum(vs)
            cnt2, last2 = plsc.scan_count(ks, mask=m2)
            um2 = last2 & m2
            aout[pl.ds(NB, 16)] = csum
            ip = i16 - cnt2
            prev = plsc.load_gather(aout, [jnp.maximum(ip, 0) + NB])
            prev = jnp.where(ip >= 0, prev, z16f)
            gsum = csum - prev
            plsc.addupdate_scatter(table, [jnp.where(m2, ks, 0)], gsum,
                                   mask=um2)
            return 0

        lax.fori_loop(0, nv, pass_b, 0)

        def pass_c(t, accs):
            amx, amn, act, asm = accs
            off = s0 + t * 16
            m = i16 < (ln - t * 16)
            s = jnp.where(m, ss[pl.ds(off, 16)], z16i)
            g = plsc.load_gather(table, [s], mask=m)
            cl = plsc.load_gather(claim, [s], mask=m)
            rep = (cl == (off + i16)) & m
            amx = jnp.where(rep, jnp.maximum(amx, g), amx)
            amn = jnp.where(rep, jnp.minimum(amn, g), amn)
            act = act + jnp.where(rep, jnp.ones((16,), jnp.float32), z16f)
            asm = asm + jnp.where(rep, g, z16f)
            return amx, amn, act, asm

        amx0 = jnp.full((16,), -jnp.inf, jnp.float32)
        amn0 = jnp.full((16,), jnp.inf, jnp.float32)
        amx, amn, act, asm = lax.fori_loop(
            0, nv, pass_c, (amx0, amn0, z16f, z16f))

        mx = jnp.full((16,), jnp.max(amx), jnp.float32)
        mn = jnp.full((16,), jnp.min(amn), jnp.float32)
        ct = jnp.full((16,), jnp.sum(act), jnp.float32)
        sm = jnp.full((16,), jnp.sum(asm), jnp.float32)
        empty = ct == 0.0
        mx = jnp.where(empty, 0.0, mx)
        mn = jnp.where(empty, 0.0, mn)
        mean = sm / jnp.maximum(ct, 1.0)
        aval = (w0 * ic(mx) + w1 * ic(mean) + w2 * ic(mn) + w3 * ic(sm)
                + ba2)
        plsc.store_scatter(aout, [jnp.full((16,), nr, jnp.int32)],
                           aval, mask=i16 == 0)
        return 0

    lax.fori_loop(0, NB, node_body, 0)
    pltpu.sync_copy(aout.at[pl.ds(0, NB)], a_out_hbm.at[pl.ds(base, NB)])


def _final_body(dst_hbm, e_hbm, a_hbm, wa_hbm, out_hbm,
                av, db, ebuf, ob, wav):
    i16 = lax.iota(jnp.int32, 16)
    wid = lax.axis_index("s") * 2 + lax.axis_index("c")
    ebase = wid * EPT
    pltpu.sync_copy(wa_hbm, wav)
    wa = wav[...]
    wl = _lane(wa, 5)
    bl = _lane(wa, 6)
    pltpu.sync_copy(a_hbm, av)

    def chunk(c, _):
        off = ebase + c * CH3
        pltpu.sync_copy(dst_hbm.at[pl.ds(off, CH3)], db)
        pltpu.sync_copy(e_hbm.at[pl.ds(off, CH3)], ebuf)

        def body(i, _):
            d = db[pl.ds(i * 16, 16)]
            ev = ebuf[pl.ds(i * 16, 16)]
            a = plsc.load_gather(av, [d])
            e2 = jnp.where(ev == jnp.inf, 1.0, ev)
            e2 = jnp.where(jnp.isnan(e2), 1.0, e2)
            o = (e2 * wl + bl) + a * e2
            o = jnp.where(o == jnp.inf, 1.0, o)
            ob[pl.ds(i * 16, 16)] = o
            return 0

        lax.fori_loop(0, CH3 // 16, body, 0)
        pltpu.sync_copy(ob, out_hbm.at[pl.ds(off, CH3)])
        return 0

    lax.fori_loop(0, EPT // CH3, chunk, 0)


def _sc_mesh():
    return plsc.VectorSubcoreMesh(core_axis_name="c", subcore_axis_name="s")


def kernel(x, adjs, edge_attr, W_e, b_e, W_a2, b_a2, W_l, b_l):
    del x
    e = _edge_scalar(edge_attr, W_e, b_e)
    wa = jnp.concatenate([
        W_a2[0], b_a2, W_l[0], b_l,
        jnp.zeros((9,), jnp.float32)]).astype(jnp.float32)

    graph = pl.kernel(
        _graph_body,
        out_type=jax.ShapeDtypeStruct((NPAD,), jnp.float32),
        mesh=_sc_mesh(),
        compiler_params=pltpu.CompilerParams(needs_layout_passes=False),
        scratch_types=[
            pltpu.VMEM((CH,), jnp.int32),
            pltpu.VMEM((CH,), jnp.int32),
            pltpu.VMEM((CH,), jnp.float32),
            pltpu.VMEM((CAPP,), jnp.int32),
            pltpu.VMEM((CAPP,), jnp.int32),
            pltpu.VMEM((CAPP,), jnp.float32),
            pltpu.VMEM((CAPS,), jnp.int32),
            pltpu.VMEM((CAPS,), jnp.float32),
            pltpu.VMEM((HB,), jnp.float32),
            pltpu.VMEM((HB,), jnp.float32),
            pltpu.VMEM((HB,), jnp.float32),
            pltpu.VMEM((16,), jnp.float32),
            pltpu.VMEM((NB + 16,), jnp.float32),
        ],
    )
    A = graph(adjs[0], adjs[1], e, wa)

    final = pl.kernel(
        _final_body,
        out_type=jax.ShapeDtypeStruct((E,), jnp.float32),
        mesh=_sc_mesh(),
        compiler_params=pltpu.CompilerParams(needs_layout_passes=False),
        scratch_types=[
            pltpu.VMEM((NPAD,), jnp.float32),
            pltpu.VMEM((CH3,), jnp.int32),
            pltpu.VMEM((CH3,), jnp.float32),
            pltpu.VMEM((CH3,), jnp.float32),
            pltpu.VMEM((16,), jnp.float32),
        ],
    )
    return final(adjs[1], e, A, wa)


# speedup vs baseline: 13.7276x; 1.1398x over previous
import functools

import jax
import jax.numpy as jnp
from jax import lax
from jax.experimental import pallas as pl
from jax.experimental.pallas import tpu as pltpu
from jax.experimental.pallas import tpu_sc as plsc

N = 10000
E = 320000
DF = 16

NW = 32
NB = 320
NPAD = NW * NB
CH = 3200
NCHUNK = E // CH
CAP = 21504
CAPP = CAP + 16
CAPS = CAP + 48
HB = NB + 16
CH3 = 2000
EPT = E // NW

_IOTA = None


def _lane(v, k):
    i = lax.iota(jnp.int32, 16)
    z = jnp.zeros((16,), v.dtype)
    return jnp.sum(jnp.where(i == k, v, z))


def _matmul_body(x_ref, m_ref, b_ref, o_ref):
    o_ref[...] = jnp.dot(x_ref[...], m_ref[...],
                         preferred_element_type=jnp.float32) + b_ref[0, 0]


def _edge_scalar(edge_attr, W_e, b_e):
    j = lax.broadcasted_iota(jnp.int32, (2048, 128), 0)
    l = lax.broadcasted_iota(jnp.int32, (2048, 128), 1)
    M = jnp.where(j // DF == l, W_e[0][j % DF], 0.0)
    xv = edge_attr.reshape(2500, 2048)
    b = b_e.reshape(1, 1)
    out = pl.pallas_call(
        _matmul_body,
        in_specs=[
            pl.BlockSpec((2500, 2048), lambda: (0, 0)),
            pl.BlockSpec((2048, 128), lambda: (0, 0)),
            pl.BlockSpec(memory_space=pltpu.SMEM),
        ],
        out_specs=pl.BlockSpec((2500, 128), lambda: (0, 0)),
        out_shape=jax.ShapeDtypeStruct((2500, 128), jnp.float32),
    )(xv, M, b)
    return out.reshape(E)


def _graph_body(src_hbm, dst_hbm, e_hbm, wa_hbm,
                a_out_hbm,
                a0b, a1b, eb, fd, fs, fv, ss, sv,
                hist, cursor, starts, wav, aout):
    i16 = lax.iota(jnp.int32, 16)
    z16f = jnp.zeros((16,), jnp.float32)
    z16i = jnp.zeros((16,), jnp.int32)
    wid = lax.axis_index("s") * 2 + lax.axis_index("c")
    base = wid * NB

    pltpu.sync_copy(wa_hbm, wav)
    wa = wav[...]

    def filt_chunk(c, cur):
        pltpu.sync_copy(dst_hbm.at[pl.ds(c * CH, CH)], a0b)
        pltpu.sync_copy(src_hbm.at[pl.ds(c * CH, CH)], a1b)
        pltpu.sync_copy(e_hbm.at[pl.ds(c * CH, CH)], eb)

        def body(i, cur):
            d = a0b[pl.ds(i * 16, 16)]
            s = a1b[pl.ds(i * 16, 16)]
            v = eb[pl.ds(i * 16, 16)]
            rd = d - base
            m = (rd >= 0) & (rd < NB)
            pc = jnp.sum(jnp.where(m, jnp.ones((16,), jnp.float32),
                                   z16f)).astype(jnp.int32)
            cc = jnp.minimum(cur, CAP)
            plsc.store_compressed(fd.at[pl.ds(cc, 16)], rd, mask=m)
            plsc.store_compressed(fs.at[pl.ds(cc, 16)], s, mask=m)
            plsc.store_compressed(fv.at[pl.ds(cc, 16)], v, mask=m)
            cur = cur + pc
            rd2 = s - base
            m2 = (rd2 >= 0) & (rd2 < NB)
            pc2 = jnp.sum(jnp.where(m2, jnp.ones((16,), jnp.float32),
                                    z16f)).astype(jnp.int32)
            cc2 = jnp.minimum(cur, CAP)
            plsc.store_compressed(fd.at[pl.ds(cc2, 16)], rd2, mask=m2)
            plsc.store_compressed(fs.at[pl.ds(cc2, 16)], d, mask=m2)
            plsc.store_compressed(fv.at[pl.ds(cc2, 16)], v, mask=m2)
            return cur + pc2

        return lax.fori_loop(0, CH // 16, body, cur)

    cur = lax.fori_loop(0, NCHUNK, filt_chunk, jnp.int32(0))
    cur = jnp.minimum(cur, CAP)

    fd[pl.ds(cur, 16)] = jnp.full((16,), NB, jnp.int32)
    fs[pl.ds(cur, 16)] = z16i
    fv[pl.ds(cur, 16)] = z16f
    nvec = cur // 16 + 1

    def zh(i, _):
        hist[pl.ds(i * 16, 16)] = z16f
        return 0

    lax.fori_loop(0, HB // 16, zh, 0)

    def hbody(i, _):
        rd = fd[pl.ds(i * 16, 16)]
        cnt, lastm = plsc.scan_count(rd)
        plsc.addupdate_scatter(hist, [rd], cnt.astype(jnp.float32),
                               mask=lastm)
        return 0

    lax.fori_loop(0, nvec, hbody, 0)

    def scan_body(i, carry):
        h = hist[pl.ds(i * 16, 16)]
        ex = plsc.cumsum(h) - h + carry
        cursor[pl.ds(i * 16, 16)] = ex
        starts[pl.ds(i * 16, 16)] = ex
        return carry + jnp.sum(h)

    lax.fori_loop(0, HB // 16, scan_body, jnp.float32(0))

    def pbody(i, _):
        rd = fd[pl.ds(i * 16, 16)]
        s = fs[pl.ds(i * 16, 16)]
        v = fv[pl.ds(i * 16, 16)]
        cnt, lastm = plsc.scan_count(rd)
        cntf = cnt.astype(jnp.float32)
        bpos = plsc.load_gather(cursor, [rd])
        pos = (bpos + cntf).astype(jnp.int32) - 1
        plsc.store_scatter(ss, [pos], s)
        plsc.store_scatter(sv, [pos], v)
        plsc.addupdate_scatter(cursor, [rd], cntf, mask=lastm)
        return 0

    lax.fori_loop(0, nvec, pbody, 0)

    mark, claim, table = fd, fs, fv

    def zm(i, _):
        mark[pl.ds(i * 16, 16)] = jnp.full((16,), -1, jnp.int32)
        return 0

    lax.fori_loop(0, N // 16, zm, 0)

    w0 = _lane(wa, 0)
    w1 = _lane(wa, 1)
    w2 = _lane(wa, 2)
    w3 = _lane(wa, 3)
    ba2 = _lane(wa, 4)

    def ic(x):
        r = 1.0 / x
        r = jnp.where(jnp.isnan(r), 1.0, r)
        r = jnp.where(r == jnp.inf, 1.0, r)
        return r

    def node_body(nr, _):
        stv = starts[pl.ds(nr, 16)]
        s0 = _lane(stv, 0).astype(jnp.int32)
        s1 = _lane(stv, 1).astype(jnp.int32)
        ln = s1 - s0
        nv = (ln + 15) // 16
        nabs = base + nr

        def pass_a(t, _):
            off = s0 + t * 16
            m = i16 < (ln - t * 16)
            s = jnp.where(m, ss[pl.ds(off, 16)], z16i)
            _, lastm = plsc.scan_count(s, mask=m)
            um = lastm & m
            mk = plsc.load_gather(mark, [s], mask=um)
            stale = (mk != nabs) & um
            plsc.store_scatter(mark, [s], jnp.full((16,), nabs, jnp.int32),
                               mask=stale)
            plsc.store_scatter(table, [s], z16f, mask=stale)
            plsc.store_scatter(claim, [s], off + i16, mask=um)
            return 0

        lax.fori_loop(0, nv, pass_a, 0)

        def pass_b(t, _):
            off = s0 + t * 16
            m = i16 < (ln - t * 16)
            s = jnp.where(m, ss[pl.ds(off, 16)], z16i)
            v = jnp.where(m, sv[pl.ds(off, 16)], z16f)
            ks, vs, m2 = plsc.sort_key_val(s, v, mask=m)
            vs = jnp.where(m2, vs, z16f)
            csum = plsc.cumsum(vs)
            cnt2, last2 = plsc.scan_count(ks, mask=m2)
            um2 = last2 & m2
            aout[pl.ds(NB, 16)] = csum
            ip = i16 - cnt2
            prev = plsc.load_gather(aout, [jnp.maximum(ip, 0) + NB])
            prev = jnp.where(ip >= 0, prev, z16f)
            gsum = csum - prev
            plsc.addupdate_scatter(table, [jnp.where(m2, ks, 0)], gsum,
                                   mask=um2)
            return 0

        lax.fori_loop(0, nv, pass_b, 0)

        def pass_c(t, accs):
            amx, amn, act, asm = accs
            off = s0 + t * 16
            m = i16 < (ln - t * 16)
            s = jnp.where(m, ss[pl.ds(off, 16)], z16i)
            g = plsc.load_gather(table, [s], mask=m)
            cl = plsc.load_gather(claim, [s], mask=m)
            rep = (cl == (off + i16)) & m
            amx = jnp.where(rep, jnp.maximum(amx, g), amx)
            amn = jnp.where(rep, jnp.minimum(amn, g), amn)
            act = act + jnp.where(rep, jnp.ones((16,), jnp.float32), z16f)
            asm = asm + jnp.where(rep, g, z16f)
            return amx, amn, act, asm

        amx0 = jnp.full((16,), -jnp.inf, jnp.float32)
        amn0 = jnp.full((16,), jnp.inf, jnp.float32)
        amx, amn, act, asm = lax.fori_loop(
            0, nv, pass_c, (amx0, amn0, z16f, z16f))

        mx = jnp.full((16,), jnp.max(amx), jnp.float32)
        mn = jnp.full((16,), jnp.min(amn), jnp.float32)
        ct = jnp.full((16,), jnp.sum(act), jnp.float32)
        sm = jnp.full((16,), jnp.sum(asm), jnp.float32)
        empty = ct == 0.0
        mx = jnp.where(empty, 0.0, mx)
        mn = jnp.where(empty, 0.0, mn)
        mean = sm / jnp.maximum(ct, 1.0)
        aval = (w0 * ic(mx) + w1 * ic(mean) + w2 * ic(mn) + w3 * ic(sm)
                + ba2)
        plsc.store_scatter(aout, [jnp.full((16,), nr, jnp.int32)],
                           aval, mask=i16 == 0)
        return 0

    lax.fori_loop(0, NB, node_body, 0)
    pltpu.sync_copy(aout.at[pl.ds(0, NB)], a_out_hbm.at[pl.ds(base, NB)])


def _final_body(dst_hbm, e_hbm, a_hbm, wa_hbm, out_hbm,
                av, db, ebuf, ob, wav):
    i16 = lax.iota(jnp.int32, 16)
    wid = lax.axis_index("s") * 2 + lax.axis_index("c")
    ebase = wid * EPT
    pltpu.sync_copy(wa_hbm, wav)
    wa = wav[...]
    wl = _lane(wa, 5)
    bl = _lane(wa, 6)
    pltpu.sync_copy(a_hbm, av)

    def chunk(c, _):
        off = ebase + c * CH3
        pltpu.sync_copy(dst_hbm.at[pl.ds(off, CH3)], db)
        pltpu.sync_copy(e_hbm.at[pl.ds(off, CH3)], ebuf)

        def body(i, _):
            d = db[pl.ds(i * 16, 16)]
            ev = ebuf[pl.ds(i * 16, 16)]
            a = plsc.load_gather(av, [d])
            e2 = jnp.where(ev == jnp.inf, 1.0, ev)
            e2 = jnp.where(jnp.isnan(e2), 1.0, e2)
            o = (e2 * wl + bl) + a * e2
            o = jnp.where(o == jnp.inf, 1.0, o)
            ob[pl.ds(i * 16, 16)] = o
            return 0

        lax.fori_loop(0, CH3 // 16, body, 0)
        pltpu.sync_copy(ob, out_hbm.at[pl.ds(off, CH3)])
        return 0

    lax.fori_loop(0, EPT // CH3, chunk, 0)


def _sc_mesh():
    return plsc.VectorSubcoreMesh(core_axis_name="c", subcore_axis_name="s")


def kernel(x, adjs, edge_attr, W_e, b_e, W_a2, b_a2, W_l, b_l):
    del x
    e = _edge_scalar(edge_attr, W_e, b_e)
    wa = jnp.concatenate([
        W_a2[0], b_a2, W_l[0], b_l,
        jnp.zeros((9,), jnp.float32)]).astype(jnp.float32)

    graph = pl.kernel(
        _graph_body,
        out_type=jax.ShapeDtypeStruct((NPAD,), jnp.float32),
        mesh=_sc_mesh(),
        compiler_params=pltpu.CompilerParams(needs_layout_passes=False),
        scratch_types=[
            pltpu.VMEM((CH,), jnp.int32),
            pltpu.VMEM((CH,), jnp.int32),
            pltpu.VMEM((CH,), jnp.float32),
            pltpu.VMEM((CAPP,), jnp.int32),
            pltpu.VMEM((CAPP,), jnp.int32),
            pltpu.VMEM((CAPP,), jnp.float32),
            pltpu.VMEM((CAPS,), jnp.int32),
            pltpu.VMEM((CAPS,), jnp.float32),
            pltpu.VMEM((HB,), jnp.float32),
            pltpu.VMEM((HB,), jnp.float32),
            pltpu.VMEM((HB,), jnp.float32),
            pltpu.VMEM((16,), jnp.float32),
            pltpu.VMEM((NB + 16,), jnp.float32),
        ],
    )
    A = graph(adjs[0], adjs[1], e, wa)

    final = pl.kernel(
        _final_body,
        out_type=jax.ShapeDtypeStruct((E,), jnp.float32),
        mesh=_sc_mesh(),
        compiler_params=pltpu.CompilerParams(needs_layout_passes=False),
        scratch_types=[
            pltpu.VMEM((NPAD,), jnp.float32),
            pltpu.VMEM((CH3,), jnp.int32),
            pltpu.VMEM((CH3,), jnp.float32),
            pltpu.VMEM((CH3,), jnp.float32),
            pltpu.VMEM((16,), jnp.float32),
        ],
    )
    return final(adjs[1], e, A, wa)


# speedup vs baseline: 16.7795x; 1.2223x over previous
import functools

import jax
import jax.numpy as jnp
from jax import lax
from jax.experimental import pallas as pl
from jax.experimental.pallas import tpu as pltpu
from jax.experimental.pallas import tpu_sc as plsc

N = 10000
E = 320000
DF = 16

NW = 32
NB = 320
NPAD = NW * NB
CH = 3200
NCHUNK = E // CH
CHP = 3 * CH
CAP = 21504
CAPP = CAP + 16
CAPS = CAP + 48
HB = NB + 16
CH3 = 2000
EPT = E // NW

_IOTA = None


def _lane(v, k):
    i = lax.iota(jnp.int32, 16)
    z = jnp.zeros((16,), v.dtype)
    return jnp.sum(jnp.where(i == k, v, z))


def _matmul_body(x_ref, m_ref, b_ref, o_ref):
    o_ref[...] = jnp.dot(x_ref[...], m_ref[...],
                         preferred_element_type=jnp.float32) + b_ref[0, 0]


def _edge_scalar(edge_attr, W_e, b_e):
    j = lax.broadcasted_iota(jnp.int32, (2048, 128), 0)
    l = lax.broadcasted_iota(jnp.int32, (2048, 128), 1)
    M = jnp.where(j // DF == l, W_e[0][j % DF], 0.0)
    xv = edge_attr.reshape(2500, 2048)
    b = b_e.reshape(1, 1)
    out = pl.pallas_call(
        _matmul_body,
        in_specs=[
            pl.BlockSpec((2500, 2048), lambda: (0, 0)),
            pl.BlockSpec((2048, 128), lambda: (0, 0)),
            pl.BlockSpec(memory_space=pltpu.SMEM),
        ],
        out_specs=pl.BlockSpec((2500, 128), lambda: (0, 0)),
        out_shape=jax.ShapeDtypeStruct((2500, 128), jnp.float32),
    )(xv, M, b)
    return out.reshape(E)


def _graph_body(packed_hbm, wa_hbm,
                a_out_hbm,
                fd, fs, fv, ss, sv,
                hist, cursor, starts, wav, aout, sem0, sem1):
    i16 = lax.iota(jnp.int32, 16)
    z16f = jnp.zeros((16,), jnp.float32)
    z16i = jnp.zeros((16,), jnp.int32)
    wid = lax.axis_index("s") * 2 + lax.axis_index("c")
    base = wid * NB

    pltpu.sync_copy(wa_hbm, wav)
    wa = wav[...]

    def issue(c, off, sem):
        pltpu.async_copy(packed_hbm.at[pl.ds(c * CHP, CHP)],
                         ss.at[pl.ds(off, CHP)], sem)

    def drain(off, sem):
        pltpu.make_async_copy(packed_hbm.at[pl.ds(0, CHP)],
                              ss.at[pl.ds(off, CHP)], sem).wait()

    def process(off, cur):
        def body(i, cur):
            d = ss[pl.ds(off + i * 16, 16)]
            s = ss[pl.ds(off + CH + i * 16, 16)]
            v = plsc.bitcast(ss[pl.ds(off + 2 * CH + i * 16, 16)],
                             jnp.float32)
            rd = d - base
            m = (rd >= 0) & (rd < NB)
            pc = jnp.sum(jnp.where(m, jnp.ones((16,), jnp.float32),
                                   z16f)).astype(jnp.int32)
            cc = jnp.minimum(cur, CAP)
            plsc.store_compressed(fd.at[pl.ds(cc, 16)], rd, mask=m)
            plsc.store_compressed(fs.at[pl.ds(cc, 16)], s, mask=m)
            plsc.store_compressed(fv.at[pl.ds(cc, 16)], v, mask=m)
            cur = cur + pc
            rd2 = s - base
            m2 = (rd2 >= 0) & (rd2 < NB)
            pc2 = jnp.sum(jnp.where(m2, jnp.ones((16,), jnp.float32),
                                    z16f)).astype(jnp.int32)
            cc2 = jnp.minimum(cur, CAP)
            plsc.store_compressed(fd.at[pl.ds(cc2, 16)], rd2, mask=m2)
            plsc.store_compressed(fs.at[pl.ds(cc2, 16)], d, mask=m2)
            plsc.store_compressed(fv.at[pl.ds(cc2, 16)], v, mask=m2)
            return cur + pc2

        return lax.fori_loop(0, CH // 16, body, cur)

    issue(0, 0, sem0)

    def chunk2(c2, cur):
        c = c2 * 2
        issue(c + 1, CHP, sem1)
        drain(0, sem0)
        cur = process(0, cur)

        @pl.when(c2 + 1 < NCHUNK // 2)
        def _():
            issue(c + 2, 0, sem0)

        drain(CHP, sem1)
        return process(CHP, cur)

    cur = lax.fori_loop(0, NCHUNK // 2, chunk2, jnp.int32(0))
    cur = jnp.minimum(cur, CAP)

    fd[pl.ds(cur, 16)] = jnp.full((16,), NB, jnp.int32)
    fs[pl.ds(cur, 16)] = z16i
    fv[pl.ds(cur, 16)] = z16f
    nvec = cur // 16 + 1

    def zh(i, _):
        hist[pl.ds(i * 16, 16)] = z16f
        return 0

    lax.fori_loop(0, HB // 16, zh, 0)

    def hbody(i, _):
        rd = fd[pl.ds(i * 16, 16)]
        cnt, lastm = plsc.scan_count(rd)
        plsc.addupdate_scatter(hist, [rd], cnt.astype(jnp.float32),
                               mask=lastm)
        return 0

    lax.fori_loop(0, nvec, hbody, 0)

    def scan_body(i, carry):
        h = hist[pl.ds(i * 16, 16)]
        ex = plsc.cumsum(h) - h + carry
        cursor[pl.ds(i * 16, 16)] = ex
        starts[pl.ds(i * 16, 16)] = ex
        return carry + jnp.sum(h)

    lax.fori_loop(0, HB // 16, scan_body, jnp.float32(0))

    def pbody(i, _):
        rd = fd[pl.ds(i * 16, 16)]
        s = fs[pl.ds(i * 16, 16)]
        v = fv[pl.ds(i * 16, 16)]
        cnt, lastm = plsc.scan_count(rd)
        cntf = cnt.astype(jnp.float32)
        bpos = plsc.load_gather(cursor, [rd])
        pos = (bpos + cntf).astype(jnp.int32) - 1
        plsc.store_scatter(ss, [pos], s)
        plsc.store_scatter(sv, [pos], v)
        plsc.addupdate_scatter(cursor, [rd], cntf, mask=lastm)
        return 0

    lax.fori_loop(0, nvec, pbody, 0)

    mark, claim, table = fd, fs, fv

    def zm(i, _):
        mark[pl.ds(i * 16, 16)] = jnp.full((16,), -1, jnp.int32)
        return 0

    lax.fori_loop(0, N // 16, zm, 0)

    w0 = _lane(wa, 0)
    w1 = _lane(wa, 1)
    w2 = _lane(wa, 2)
    w3 = _lane(wa, 3)
    ba2 = _lane(wa, 4)

    def ic(x):
        r = 1.0 / x
        r = jnp.where(jnp.isnan(r), 1.0, r)
        r = jnp.where(r == jnp.inf, 1.0, r)
        return r

    def node_body(nr, _):
        stv = starts[pl.ds(nr, 16)]
        s0 = _lane(stv, 0).astype(jnp.int32)
        s1 = _lane(stv, 1).astype(jnp.int32)
        ln = s1 - s0
        nv = (ln + 15) // 16
        nabs = base + nr

        def pass_a(t, _):
            off = s0 + t * 16
            m = i16 < (ln - t * 16)
            s = jnp.where(m, ss[pl.ds(off, 16)], z16i)
            _, lastm = plsc.scan_count(s, mask=m)
            um = lastm & m
            mk = plsc.load_gather(mark, [s], mask=um)
            stale = (mk != nabs) & um
            plsc.store_scatter(mark, [s], jnp.full((16,), nabs, jnp.int32),
                               mask=stale)
            plsc.store_scatter(table, [s], z16f, mask=stale)
            plsc.store_scatter(claim, [s], off + i16, mask=um)
            return 0

        lax.fori_loop(0, nv, pass_a, 0)

        def pass_b(t, _):
            off = s0 + t * 16
            m = i16 < (ln - t * 16)
            s = jnp.where(m, ss[pl.ds(off, 16)], z16i)
            v = jnp.where(m, sv[pl.ds(off, 16)], z16f)
            ks, vs, m2 = plsc.sort_key_val(s, v, mask=m)
            vs = jnp.where(m2, vs, z16f)
            csum = plsc.cumsum(vs)
            cnt2, last2 = plsc.scan_count(ks, mask=m2)
            um2 = last2 & m2
            aout[pl.ds(NB, 16)] = csum
            ip = i16 - cnt2
            prev = plsc.load_gather(aout, [jnp.maximum(ip, 0) + NB])
            prev = jnp.where(ip >= 0, prev, z16f)
            gsum = csum - prev
            plsc.addupdate_scatter(table, [jnp.where(m2, ks, 0)], gsum,
                                   mask=um2)
            return 0

        lax.fori_loop(0, nv, pass_b, 0)

        def pass_c(t, accs):
            amx, amn, act, asm = accs
            off = s0 + t * 16
            m = i16 < (ln - t * 16)
            s = jnp.where(m, ss[pl.ds(off, 16)], z16i)
            g = plsc.load_gather(table, [s], mask=m)
            cl = plsc.load_gather(claim, [s], mask=m)
            rep = (cl == (off + i16)) & m
            amx = jnp.where(rep, jnp.maximum(amx, g), amx)
            amn = jnp.where(rep, jnp.minimum(amn, g), amn)
            act = act + jnp.where(rep, jnp.ones((16,), jnp.float32), z16f)
            asm = asm + jnp.where(rep, g, z16f)
            return amx, amn, act, asm

        amx0 = jnp.full((16,), -jnp.inf, jnp.float32)
        amn0 = jnp.full((16,), jnp.inf, jnp.float32)
        amx, amn, act, asm = lax.fori_loop(
            0, nv, pass_c, (amx0, amn0, z16f, z16f))

        mx = jnp.full((16,), jnp.max(amx), jnp.float32)
        mn = jnp.full((16,), jnp.min(amn), jnp.float32)
        ct = jnp.full((16,), jnp.sum(act), jnp.float32)
        sm = jnp.full((16,), jnp.sum(asm), jnp.float32)
        empty = ct == 0.0
        mx = jnp.where(empty, 0.0, mx)
        mn = jnp.where(empty, 0.0, mn)
        mean = sm / jnp.maximum(ct, 1.0)
        aval = (w0 * ic(mx) + w1 * ic(mean) + w2 * ic(mn) + w3 * ic(sm)
                + ba2)
        plsc.store_scatter(aout, [jnp.full((16,), nr, jnp.int32)],
                           aval, mask=i16 == 0)
        return 0

    lax.fori_loop(0, NB, node_body, 0)
    pltpu.sync_copy(aout.at[pl.ds(0, NB)], a_out_hbm.at[pl.ds(base, NB)])


def _final_body(dst_hbm, e_hbm, a_hbm, wa_hbm, out_hbm,
                av, db, ebuf, ob, wav):
    i16 = lax.iota(jnp.int32, 16)
    wid = lax.axis_index("s") * 2 + lax.axis_index("c")
    ebase = wid * EPT
    pltpu.sync_copy(wa_hbm, wav)
    wa = wav[...]
    wl = _lane(wa, 5)
    bl = _lane(wa, 6)
    pltpu.sync_copy(a_hbm, av)

    def chunk(c, _):
        off = ebase + c * CH3
        pltpu.sync_copy(dst_hbm.at[pl.ds(off, CH3)], db)
        pltpu.sync_copy(e_hbm.at[pl.ds(off, CH3)], ebuf)

        def body(i, _):
            d = db[pl.ds(i * 16, 16)]
            ev = ebuf[pl.ds(i * 16, 16)]
            a = plsc.load_gather(av, [d])
            e2 = jnp.where(ev == jnp.inf, 1.0, ev)
            e2 = jnp.where(jnp.isnan(e2), 1.0, e2)
            o = (e2 * wl + bl) + a * e2
            o = jnp.where(o == jnp.inf, 1.0, o)
            ob[pl.ds(i * 16, 16)] = o
            return 0

        lax.fori_loop(0, CH3 // 16, body, 0)
        pltpu.sync_copy(ob, out_hbm.at[pl.ds(off, CH3)])
        return 0

    lax.fori_loop(0, EPT // CH3, chunk, 0)


def _sc_mesh():
    return plsc.VectorSubcoreMesh(core_axis_name="c", subcore_axis_name="s")


def kernel(x, adjs, edge_attr, W_e, b_e, W_a2, b_a2, W_l, b_l):
    del x
    e = _edge_scalar(edge_attr, W_e, b_e)
    wa = jnp.concatenate([
        W_a2[0], b_a2, W_l[0], b_l,
        jnp.zeros((9,), jnp.float32)]).astype(jnp.float32)

    ebits = lax.bitcast_convert_type(e, jnp.int32)
    packed = jnp.concatenate(
        [adjs[1].reshape(NCHUNK, CH), adjs[0].reshape(NCHUNK, CH),
         ebits.reshape(NCHUNK, CH)], axis=1).reshape(-1)

    graph = pl.kernel(
        _graph_body,
        out_type=jax.ShapeDtypeStruct((NPAD,), jnp.float32),
        mesh=_sc_mesh(),
        compiler_params=pltpu.CompilerParams(needs_layout_passes=False),
        scratch_types=[
            pltpu.VMEM((CAPP,), jnp.int32),
            pltpu.VMEM((CAPP,), jnp.int32),
            pltpu.VMEM((CAPP,), jnp.float32),
            pltpu.VMEM((CAPS,), jnp.int32),
            pltpu.VMEM((CAPS,), jnp.float32),
            pltpu.VMEM((HB,), jnp.float32),
            pltpu.VMEM((HB,), jnp.float32),
            pltpu.VMEM((HB,), jnp.float32),
            pltpu.VMEM((16,), jnp.float32),
            pltpu.VMEM((NB + 16,), jnp.float32),
            pltpu.SemaphoreType.DMA,
            pltpu.SemaphoreType.DMA,
        ],
    )
    A = graph(packed, wa)

    final = pl.kernel(
        _final_body,
        out_type=jax.ShapeDtypeStruct((E,), jnp.float32),
        mesh=_sc_mesh(),
        compiler_params=pltpu.CompilerParams(needs_layout_passes=False),
        scratch_types=[
            pltpu.VMEM((NPAD,), jnp.float32),
            pltpu.VMEM((CH3,), jnp.int32),
            pltpu.VMEM((CH3,), jnp.float32),
            pltpu.VMEM((CH3,), jnp.float32),
            pltpu.VMEM((16,), jnp.float32),
        ],
    )
    return final(adjs[1], e, A, wa)


# speedup vs baseline: 17.3956x; 1.0367x over previous
import functools

import jax
import jax.numpy as jnp
from jax import lax
from jax.experimental import pallas as pl
from jax.experimental.pallas import tpu as pltpu
from jax.experimental.pallas import tpu_sc as plsc

N = 10000
E = 320000
DF = 16

NW = 32
NB = 320
NPAD = NW * NB
CH = 3200
NCHUNK = E // CH
CHP = 3 * CH
CAP = 21504
CAPP = CAP + 16
CAPS = CAP + 48
HB = NB + 16
CH3 = 2000
EPT = E // NW

_IOTA = None


def _lane(v, k):
    i = lax.iota(jnp.int32, 16)
    z = jnp.zeros((16,), v.dtype)
    return jnp.sum(jnp.where(i == k, v, z))


def _matmul_body(x_ref, m_ref, b_ref, o_ref):
    o_ref[...] = jnp.dot(x_ref[...], m_ref[...],
                         preferred_element_type=jnp.float32) + b_ref[0, 0]


def _edge_scalar(edge_attr, W_e, b_e):
    j = lax.broadcasted_iota(jnp.int32, (2048, 128), 0)
    l = lax.broadcasted_iota(jnp.int32, (2048, 128), 1)
    M = jnp.where(j // DF == l, W_e[0][j % DF], 0.0)
    xv = edge_attr.reshape(2500, 2048)
    b = b_e.reshape(1, 1)
    out = pl.pallas_call(
        _matmul_body,
        in_specs=[
            pl.BlockSpec((2500, 2048), lambda: (0, 0)),
            pl.BlockSpec((2048, 128), lambda: (0, 0)),
            pl.BlockSpec(memory_space=pltpu.SMEM),
        ],
        out_specs=pl.BlockSpec((2500, 128), lambda: (0, 0)),
        out_shape=jax.ShapeDtypeStruct((2500, 128), jnp.float32),
    )(xv, M, b)
    return out.reshape(E)


def _graph_body(packed_hbm, wa_hbm,
                a_out_hbm,
                fd, fs, fv, ss, sv,
                hist, cursor, starts, wav, aout, sem0, sem1):
    i16 = lax.iota(jnp.int32, 16)
    z16f = jnp.zeros((16,), jnp.float32)
    z16i = jnp.zeros((16,), jnp.int32)
    wid = lax.axis_index("s") * 2 + lax.axis_index("c")
    base = wid * NB

    pltpu.sync_copy(wa_hbm, wav)
    wa = wav[...]

    def issue(c, off, sem):
        pltpu.async_copy(packed_hbm.at[pl.ds(c * CHP, CHP)],
                         ss.at[pl.ds(off, CHP)], sem)

    def drain(off, sem):
        pltpu.make_async_copy(packed_hbm.at[pl.ds(0, CHP)],
                              ss.at[pl.ds(off, CHP)], sem).wait()

    def process(off, cur):
        def half(o, cur):
            d = ss[pl.ds(o, 16)]
            s = ss[pl.ds(o + CH, 16)]
            v = plsc.bitcast(ss[pl.ds(o + 2 * CH, 16)], jnp.float32)
            rd = d - base
            m = plsc.bitcast(rd, jnp.uint32) < jnp.uint32(NB)
            pc = jnp.sum(jnp.where(m, jnp.ones((16,), jnp.float32),
                                   z16f)).astype(jnp.int32)
            cc = jnp.minimum(cur, CAP)
            plsc.store_compressed(fd.at[pl.ds(cc, 16)], rd, mask=m)
            plsc.store_compressed(fs.at[pl.ds(cc, 16)], s, mask=m)
            plsc.store_compressed(fv.at[pl.ds(cc, 16)], v, mask=m)
            cur = cur + pc
            rd2 = s - base
            m2 = plsc.bitcast(rd2, jnp.uint32) < jnp.uint32(NB)
            pc2 = jnp.sum(jnp.where(m2, jnp.ones((16,), jnp.float32),
                                    z16f)).astype(jnp.int32)
            cc2 = jnp.minimum(cur, CAP)
            plsc.store_compressed(fd.at[pl.ds(cc2, 16)], rd2, mask=m2)
            plsc.store_compressed(fs.at[pl.ds(cc2, 16)], d, mask=m2)
            plsc.store_compressed(fv.at[pl.ds(cc2, 16)], v, mask=m2)
            return cur + pc2

        def body(i, cur):
            o = off + i * 32
            cur = half(o, cur)
            return half(o + 16, cur)

        return lax.fori_loop(0, CH // 32, body, cur)

    issue(0, 0, sem0)

    def chunk2(c2, cur):
        c = c2 * 2
        issue(c + 1, CHP, sem1)
        drain(0, sem0)
        cur = process(0, cur)

        @pl.when(c2 + 1 < NCHUNK // 2)
        def _():
            issue(c + 2, 0, sem0)

        drain(CHP, sem1)
        return process(CHP, cur)

    cur = lax.fori_loop(0, NCHUNK // 2, chunk2, jnp.int32(0))
    cur = jnp.minimum(cur, CAP)

    fd[pl.ds(cur, 16)] = jnp.full((16,), NB, jnp.int32)
    fs[pl.ds(cur, 16)] = z16i
    fv[pl.ds(cur, 16)] = z16f
    nvec = cur // 16 + 1

    def zh(i, _):
        hist[pl.ds(i * 16, 16)] = z16f
        return 0

    lax.fori_loop(0, HB // 16, zh, 0)

    def hbody(i, _):
        rd = fd[pl.ds(i * 16, 16)]
        cnt, lastm = plsc.scan_count(rd)
        plsc.addupdate_scatter(hist, [rd], cnt.astype(jnp.float32),
                               mask=lastm)
        return 0

    lax.fori_loop(0, nvec, hbody, 0)

    def scan_body(i, carry):
        h = hist[pl.ds(i * 16, 16)]
        ex = plsc.cumsum(h) - h + carry
        cursor[pl.ds(i * 16, 16)] = ex
        starts[pl.ds(i * 16, 16)] = ex
        return carry + jnp.sum(h)

    lax.fori_loop(0, HB // 16, scan_body, jnp.float32(0))

    def pbody(i, _):
        rd = fd[pl.ds(i * 16, 16)]
        s = fs[pl.ds(i * 16, 16)]
        v = fv[pl.ds(i * 16, 16)]
        cnt, lastm = plsc.scan_count(rd)
        cntf = cnt.astype(jnp.float32)
        bpos = plsc.load_gather(cursor, [rd])
        pos = (bpos + cntf).astype(jnp.int32) - 1
        plsc.store_scatter(ss, [pos], s)
        plsc.store_scatter(sv, [pos], v)
        plsc.addupdate_scatter(cursor, [rd], cntf, mask=lastm)
        return 0

    lax.fori_loop(0, nvec, pbody, 0)

    mark, claim, table = fd, fs, fv

    def zm(i, _):
        mark[pl.ds(i * 16, 16)] = jnp.full((16,), -1, jnp.int32)
        return 0

    lax.fori_loop(0, N // 16, zm, 0)

    w0 = _lane(wa, 0)
    w1 = _lane(wa, 1)
    w2 = _lane(wa, 2)
    w3 = _lane(wa, 3)
    ba2 = _lane(wa, 4)

    def ic(x):
        r = 1.0 / x
        r = jnp.where(jnp.isnan(r), 1.0, r)
        r = jnp.where(r == jnp.inf, 1.0, r)
        return r

    def node_body(nr, _):
        stv = starts[pl.ds(nr, 16)]
        s0 = _lane(stv, 0).astype(jnp.int32)
        s1 = _lane(stv, 1).astype(jnp.int32)
        ln = s1 - s0
        nv = (ln + 15) // 16
        nabs = base + nr

        def pass_a(t, _):
            off = s0 + t * 16
            m = i16 < (ln - t * 16)
            s = jnp.where(m, ss[pl.ds(off, 16)], z16i)
            _, lastm = plsc.scan_count(s, mask=m)
            um = lastm & m
            mk = plsc.load_gather(mark, [s], mask=um)
            stale = (mk != nabs) & um
            plsc.store_scatter(mark, [s], jnp.full((16,), nabs, jnp.int32),
                               mask=stale)
            plsc.store_scatter(table, [s], z16f, mask=stale)
            plsc.store_scatter(claim, [s], off + i16, mask=um)
            return 0

        lax.fori_loop(0, nv, pass_a, 0)

        def pass_b(t, _):
            off = s0 + t * 16
            m = i16 < (ln - t * 16)
            s = jnp.where(m, ss[pl.ds(off, 16)], z16i)
            v = jnp.where(m, sv[pl.ds(off, 16)], z16f)
            ks, vs, m2 = plsc.sort_key_val(s, v, mask=m)
            vs = jnp.where(m2, vs, z16f)
            csum = plsc.cumsum(vs)
            cnt2, last2 = plsc.scan_count(ks, mask=m2)
            um2 = last2 & m2
            aout[pl.ds(NB, 16)] = csum
            ip = i16 - cnt2
            prev = plsc.load_gather(aout, [jnp.maximum(ip, 0) + NB])
            prev = jnp.where(ip >= 0, prev, z16f)
            gsum = csum - prev
            plsc.addupdate_scatter(table, [jnp.where(m2, ks, 0)], gsum,
                                   mask=um2)
            return 0

        lax.fori_loop(0, nv, pass_b, 0)

        def pass_c(t, accs):
            amx, amn, act, asm = accs
            off = s0 + t * 16
            m = i16 < (ln - t * 16)
            s = jnp.where(m, ss[pl.ds(off, 16)], z16i)
            g = plsc.load_gather(table, [s], mask=m)
            cl = plsc.load_gather(claim, [s], mask=m)
            rep = (cl == (off + i16)) & m
            amx = jnp.where(rep, jnp.maximum(amx, g), amx)
            amn = jnp.where(rep, jnp.minimum(amn, g), amn)
            act = act + jnp.where(rep, jnp.ones((16,), jnp.float32), z16f)
            asm = asm + jnp.where(rep, g, z16f)
            return amx, amn, act, asm

        amx0 = jnp.full((16,), -jnp.inf, jnp.float32)
        amn0 = jnp.full((16,), jnp.inf, jnp.float32)
        amx, amn, act, asm = lax.fori_loop(
            0, nv, pass_c, (amx0, amn0, z16f, z16f))

        mx = jnp.full((16,), jnp.max(amx), jnp.float32)
        mn = jnp.full((16,), jnp.min(amn), jnp.float32)
        ct = jnp.full((16,), jnp.sum(act), jnp.float32)
        sm = jnp.full((16,), jnp.sum(asm), jnp.float32)
        empty = ct == 0.0
        mx = jnp.where(empty, 0.0, mx)
        mn = jnp.where(empty, 0.0, mn)
        mean = sm / jnp.maximum(ct, 1.0)
        aval = (w0 * ic(mx) + w1 * ic(mean) + w2 * ic(mn) + w3 * ic(sm)
                + ba2)
        plsc.store_scatter(aout, [jnp.full((16,), nr, jnp.int32)],
                           aval, mask=i16 == 0)
        return 0

    lax.fori_loop(0, NB, node_body, 0)
    pltpu.sync_copy(aout.at[pl.ds(0, NB)], a_out_hbm.at[pl.ds(base, NB)])


def _final_body(dst_hbm, e_hbm, a_hbm, wa_hbm, out_hbm,
                av, db, ebuf, ob, wav):
    i16 = lax.iota(jnp.int32, 16)
    wid = lax.axis_index("s") * 2 + lax.axis_index("c")
    ebase = wid * EPT
    pltpu.sync_copy(wa_hbm, wav)
    wa = wav[...]
    wl = _lane(wa, 5)
    bl = _lane(wa, 6)
    pltpu.sync_copy(a_hbm, av)

    def chunk(c, _):
        off = ebase + c * CH3
        pltpu.sync_copy(dst_hbm.at[pl.ds(off, CH3)], db)
        pltpu.sync_copy(e_hbm.at[pl.ds(off, CH3)], ebuf)

        def body(i, _):
            d = db[pl.ds(i * 16, 16)]
            ev = ebuf[pl.ds(i * 16, 16)]
            a = plsc.load_gather(av, [d])
            e2 = jnp.where(ev == jnp.inf, 1.0, ev)
            e2 = jnp.where(jnp.isnan(e2), 1.0, e2)
            o = (e2 * wl + bl) + a * e2
            o = jnp.where(o == jnp.inf, 1.0, o)
            ob[pl.ds(i * 16, 16)] = o
            return 0

        lax.fori_loop(0, CH3 // 16, body, 0)
        pltpu.sync_copy(ob, out_hbm.at[pl.ds(off, CH3)])
        return 0

    lax.fori_loop(0, EPT // CH3, chunk, 0)


def _sc_mesh():
    return plsc.VectorSubcoreMesh(core_axis_name="c", subcore_axis_name="s")


def kernel(x, adjs, edge_attr, W_e, b_e, W_a2, b_a2, W_l, b_l):
    del x
    e = _edge_scalar(edge_attr, W_e, b_e)
    wa = jnp.concatenate([
        W_a2[0], b_a2, W_l[0], b_l,
        jnp.zeros((9,), jnp.float32)]).astype(jnp.float32)

    ebits = lax.bitcast_convert_type(e, jnp.int32)
    packed = jnp.concatenate(
        [adjs[1].reshape(NCHUNK, CH), adjs[0].reshape(NCHUNK, CH),
         ebits.reshape(NCHUNK, CH)], axis=1).reshape(-1)

    graph = pl.kernel(
        _graph_body,
        out_type=jax.ShapeDtypeStruct((NPAD,), jnp.float32),
        mesh=_sc_mesh(),
        compiler_params=pltpu.CompilerParams(needs_layout_passes=False),
        scratch_types=[
            pltpu.VMEM((CAPP,), jnp.int32),
            pltpu.VMEM((CAPP,), jnp.int32),
            pltpu.VMEM((CAPP,), jnp.float32),
            pltpu.VMEM((CAPS,), jnp.int32),
            pltpu.VMEM((CAPS,), jnp.float32),
            pltpu.VMEM((HB,), jnp.float32),
            pltpu.VMEM((HB,), jnp.float32),
            pltpu.VMEM((HB,), jnp.float32),
            pltpu.VMEM((16,), jnp.float32),
            pltpu.VMEM((NB + 16,), jnp.float32),
            pltpu.SemaphoreType.DMA,
            pltpu.SemaphoreType.DMA,
        ],
    )
    A = graph(packed, wa)

    final = pl.kernel(
        _final_body,
        out_type=jax.ShapeDtypeStruct((E,), jnp.float32),
        mesh=_sc_mesh(),
        compiler_params=pltpu.CompilerParams(needs_layout_passes=False),
        scratch_types=[
            pltpu.VMEM((NPAD,), jnp.float32),
            pltpu.VMEM((CH3,), jnp.int32),
            pltpu.VMEM((CH3,), jnp.float32),
            pltpu.VMEM((CH3,), jnp.float32),
            pltpu.VMEM((16,), jnp.float32),
        ],
    )
    return final(adjs[1], e, A, wa)
